# baseline (speedup 1.0000x reference)
import functools

import jax
import jax.numpy as jnp
from jax import lax
from jax.experimental import pallas as pl
from jax.experimental.pallas import tpu as pltpu
from jax.experimental.pallas import tpu_sc as plsc

N_NODES = 10000
N_PAD = 10240
D = 128

NC = 2
NS = 16
NW = NC * NS
E = 320000
EPW = E // NW
CH = 80
NGG = CH // 16
NCHUNK = EPW // CH
NG = 5
GC = 25

RPS = N_PAD // NS

_MESH = plsc.VectorSubcoreMesh(core_axis_name="c", subcore_axis_name="s")
_SC_PARAMS = pltpu.CompilerParams(needs_layout_passes=False)


def _wid():
    return lax.axis_index("s") * NC + lax.axis_index("c")


@functools.partial(
    pl.kernel,
    out_type=jax.ShapeDtypeStruct((NC, NS, RPS), jnp.float32),
    mesh=_MESH,
    compiler_params=_SC_PARAMS,
    scratch_types=[
        pltpu.VMEM((NCHUNK, CH), jnp.int32),
        pltpu.VMEM((NCHUNK, CH), jnp.float32),
        pltpu.VMEM((RPS,), jnp.float32),
        pltpu.VMEM_SHARED((N_PAD,), jnp.float32),
    ],
)
def _deg_kernel(col_hbm, ew_hbm, out_hbm, col_v, ew_v, buf_v, deg_sh):
    cid = lax.axis_index("c")
    sid = lax.axis_index("s")
    wid = _wid()

    zero16 = jnp.zeros((16,), jnp.float32)
    for i in range(RPS // 16):
        buf_v[pl.ds(i * 16, 16)] = zero16
    pltpu.sync_copy(buf_v, deg_sh.at[pl.ds(sid * RPS, RPS)])
    plsc.subcore_barrier()

    pltpu.sync_copy(col_hbm.at[wid], col_v)
    pltpu.sync_copy(ew_hbm.at[wid], ew_v)

    def chunk(t, carry):
        pltpu.sync_copy(ew_v.at[t], deg_sh.at[col_v.at[t]], add=True)
        return carry

    lax.fori_loop(0, NCHUNK, chunk, 0)
    plsc.subcore_barrier()

    pltpu.sync_copy(deg_sh.at[pl.ds(sid * RPS, RPS)], buf_v)
    pltpu.sync_copy(buf_v, out_hbm.at[cid, sid])


@functools.partial(
    pl.kernel,
    out_type=jax.ShapeDtypeStruct((NC, NS, RPS, D), jnp.float32),
    mesh=_MESH,
    compiler_params=_SC_PARAMS,
    scratch_types=[
        pltpu.VMEM((GC, CH), jnp.int32),
        pltpu.VMEM((GC, CH), jnp.int32),
        pltpu.VMEM((GC, CH), jnp.float32),
        pltpu.VMEM((CH, D), jnp.float32),
        pltpu.VMEM((CH, D), jnp.float32),
        pltpu.VMEM_SHARED((N_PAD, D), jnp.float32),
        pltpu.SemaphoreType.DMA,
        pltpu.SemaphoreType.DMA,
        pltpu.SemaphoreType.DMA,
        pltpu.SemaphoreType.DMA,
    ],
)
def _msg_kernel(h_hbm, row_hbm, col_hbm, ew_hbm, out_hbm,
                row_v, col_v, ew_v, buf_a, buf_b, acc_sh,
                gsem_a, gsem_b, ssem_a, ssem_b):
    cid = lax.axis_index("c")
    sid = lax.axis_index("s")
    wid = _wid()

    zero16 = jnp.zeros((16,), jnp.float32)

    def zrow(i, carry):
        for j in range(D // 16):
            buf_a[i, pl.ds(j * 16, 16)] = zero16
        return carry

    lax.fori_loop(0, CH, zrow, 0)
    for q in range(RPS // CH):
        pltpu.sync_copy(buf_a, acc_sh.at[pl.ds(sid * RPS + q * CH, CH)])
    plsc.subcore_barrier()

    def gather_start(t, buf, sem):
        pltpu.make_async_copy(h_hbm.at[row_v.at[t]], buf, sem).start()

    def gather_wait(t, buf, sem):
        pltpu.make_async_copy(h_hbm.at[row_v.at[t]], buf, sem).wait()

    def scatter_drain(t, buf, sem):
        pltpu.make_async_copy(buf, acc_sh.at[col_v.at[t]], sem).wait()

    def scale_part(buf, t, g0, g1):
        for gg in range(g0, g1):
            nv = ew_v[t, pl.ds(gg * 16, 16)]
            for k in range(16):
                s = nv[k]
                r = gg * 16 + k
                for j in range(D // 16):
                    sl = pl.ds(j * 16, 16)
                    buf[r, sl] = buf[r, sl] * s

    def scatter_start(t, buf, sem):
        pltpu.make_async_copy(buf, acc_sh.at[col_v.at[t]], sem).start(add=True)

    def group(g, carry):
        pltpu.sync_copy(row_hbm.at[wid, g], row_v)
        pltpu.sync_copy(col_hbm.at[wid, g], col_v)
        pltpu.sync_copy(ew_hbm.at[wid, g], ew_v)

        gather_start(0, buf_a, gsem_a)

        def chunk_ops(t, buf_x, gs_x, ss_x, buf_y, gs_y, ss_y, drain_y):
            gather_wait(t, buf_x, gs_x)
            scale_part(buf_x, t, 0, 2)
            @pl.when(drain_y)
            def _():
                scatter_drain(t, buf_y, ss_y)
            pltpu.make_async_copy(h_hbm.at[row_v.at[t + 1]], buf_y, gs_y
                                  ).start()
            scale_part(buf_x, t, 2, NGG)
            scatter_start(t, buf_x, ss_x)

        def pair(i, c2):
            t0 = 2 * i
            chunk_ops(t0, buf_a, gsem_a, ssem_a, buf_b, gsem_b, ssem_b, i > 0)
            chunk_ops(t0 + 1, buf_b, gsem_b, ssem_b, buf_a, gsem_a, ssem_a,
                      jnp.bool_(True))
            return c2

        lax.fori_loop(0, GC // 2, pair, 0)
        tl = GC - 1
        gather_wait(tl, buf_a, gsem_a)
        scale_part(buf_a, tl, 0, 2)
        scatter_drain(tl, buf_b, ssem_b)
        scale_part(buf_a, tl, 2, NGG)
        scatter_start(tl, buf_a, ssem_a)
        scatter_drain(tl, buf_a, ssem_a)
        return carry

    lax.fori_loop(0, NG, group, 0)
    plsc.subcore_barrier()

    for q in range(RPS // CH):
        pltpu.sync_copy(acc_sh.at[pl.ds(sid * RPS + q * CH, CH)], buf_a)
        pltpu.sync_copy(buf_a, out_hbm.at[cid, sid, pl.ds(q * CH, CH)])


def _dis_body(degp_ref, dis_ref):
    deg = degp_ref[0] + degp_ref[1] + 1.0
    dis_ref[...] = lax.rsqrt(deg)


def _dis_call(degp):
    degp2 = degp.reshape(NC, N_PAD)
    return pl.pallas_call(
        _dis_body,
        out_shape=jax.ShapeDtypeStruct((N_PAD,), jnp.float32),
    )(degp2)


def _matmul_body(x_ref, w_ref, dis_ref, o_ref):
    o_ref[...] = jnp.dot(x_ref[...], w_ref[...],
                         preferred_element_type=jnp.float32) * dis_ref[...]


def _matmul_call(x, w, dis_col):
    bm = 1000
    grid = N_NODES // bm
    return pl.pallas_call(
        _matmul_body,
        grid=(grid,),
        in_specs=[
            pl.BlockSpec((bm, D), lambda i: (i, 0)),
            pl.BlockSpec((D, D), lambda i: (0, 0)),
            pl.BlockSpec((bm, 1), lambda i: (i, 0)),
        ],
        out_specs=pl.BlockSpec((bm, D), lambda i: (i, 0)),
        out_shape=jax.ShapeDtypeStruct((N_NODES, D), jnp.float32),
    )(x, w, dis_col)


def _combine_mm_body(m0_ref, m1_ref, h_ref, dis_ref, b_ref, a_ref, w_ref,
                     o_ref):
    dis = dis_ref[...]
    z = (m0_ref[...] + m1_ref[...] + h_ref[...]) * dis + b_ref[...]
    z = jnp.where(z > 0, z, a_ref[...] * z)
    o_ref[...] = jnp.dot(z, w_ref[...],
                         preferred_element_type=jnp.float32) * dis


def _combine_body(m0_ref, m1_ref, h_ref, dis_ref, b_ref, a_ref, o_ref):
    z = (m0_ref[...] + m1_ref[...] + h_ref[...]) * dis_ref[...] + b_ref[...]
    o_ref[...] = jnp.where(z > 0, z, a_ref[...] * z)


def _combine_call(m0, m1, h, dis_col, b, a, w=None):
    bm = 1000
    grid = N_NODES // bm
    node_spec = pl.BlockSpec((bm, D), lambda i: (i, 0))
    vec_spec = pl.BlockSpec((1, D), lambda i: (0, 0))
    in_specs = [node_spec, node_spec, node_spec,
                pl.BlockSpec((bm, 1), lambda i: (i, 0)),
                vec_spec, vec_spec]
    args = [m0, m1, h, dis_col, b.reshape(1, D), a.reshape(1, D)]
    if w is not None:
        in_specs.append(pl.BlockSpec((D, D), lambda i: (0, 0)))
        args.append(w)
        body = _combine_mm_body
    else:
        body = _combine_body
    return pl.pallas_call(
        body,
        grid=(grid,),
        in_specs=in_specs,
        out_specs=node_spec,
        out_shape=jax.ShapeDtypeStruct((N_NODES, D), jnp.float32),
    )(*args)


def kernel(x, edge_index, edge_weight, W1, b1, a1, W2, b2, a2):
    ei = edge_index.astype(jnp.int32)
    row3 = ei[0].reshape(NW, NCHUNK, CH)
    col3 = ei[1].reshape(NW, NCHUNK, CH)
    ew3 = edge_weight.reshape(NW, NCHUNK, CH)
    row4 = row3.reshape(NW, NG, GC, CH)
    col4 = col3.reshape(NW, NG, GC, CH)
    ew4 = ew3.reshape(NW, NG, GC, CH)

    degp = _deg_kernel(col3, ew3)
    dis_flat = _dis_call(degp)
    dis_col = dis_flat[:N_NODES].reshape(N_NODES, 1)

    hd1 = _matmul_call(x, W1, dis_col)
    m1 = _msg_kernel(hd1, row4, col4, ew4).reshape(NC, N_PAD, D)[:, :N_NODES]
    hd2 = _combine_call(m1[0], m1[1], hd1, dis_col, b1, a1, w=W2)

    m2 = _msg_kernel(hd2, row4, col4, ew4).reshape(NC, N_PAD, D)[:, :N_NODES]
    out = _combine_call(m2[0], m2[1], hd2, dis_col, b2, a2)
    return out


# speedup vs baseline: 1.0409x; 1.0409x over previous
import functools

import jax
import jax.numpy as jnp
from jax import lax
from jax.experimental import pallas as pl
from jax.experimental.pallas import tpu as pltpu
from jax.experimental.pallas import tpu_sc as plsc

N_NODES = 10000
N_PAD = 10240
D = 128

NC = 2
NS = 16
NW = NC * NS
E = 320000
EPW = E // NW
CH = 80
NGG = CH // 16
NCHUNK = EPW // CH
NG = 5
GC = 25

RPS = N_PAD // NS

_MESH = plsc.VectorSubcoreMesh(core_axis_name="c", subcore_axis_name="s")
_SC_PARAMS = pltpu.CompilerParams(needs_layout_passes=False)


def _wid():
    return lax.axis_index("s") * NC + lax.axis_index("c")


@functools.partial(
    pl.kernel,
    out_type=jax.ShapeDtypeStruct((NC, NS, RPS), jnp.float32),
    mesh=_MESH,
    compiler_params=_SC_PARAMS,
    scratch_types=[
        pltpu.VMEM((NCHUNK, CH), jnp.int32),
        pltpu.VMEM((NCHUNK, CH), jnp.float32),
        pltpu.VMEM((RPS,), jnp.float32),
        pltpu.VMEM_SHARED((N_PAD,), jnp.float32),
    ],
)
def _deg_kernel(col_hbm, ew_hbm, out_hbm, col_v, ew_v, buf_v, deg_sh):
    cid = lax.axis_index("c")
    sid = lax.axis_index("s")
    wid = _wid()

    zero16 = jnp.zeros((16,), jnp.float32)
    for i in range(RPS // 16):
        buf_v[pl.ds(i * 16, 16)] = zero16
    pltpu.sync_copy(buf_v, deg_sh.at[pl.ds(sid * RPS, RPS)])
    plsc.subcore_barrier()

    pltpu.sync_copy(col_hbm.at[wid], col_v)
    pltpu.sync_copy(ew_hbm.at[wid], ew_v)

    def chunk(t, carry):
        pltpu.sync_copy(ew_v.at[t], deg_sh.at[col_v.at[t]], add=True)
        return carry

    lax.fori_loop(0, NCHUNK, chunk, 0)
    plsc.subcore_barrier()

    pltpu.sync_copy(deg_sh.at[pl.ds(sid * RPS, RPS)], buf_v)
    pltpu.sync_copy(buf_v, out_hbm.at[cid, sid])


@functools.partial(
    pl.kernel,
    out_type=jax.ShapeDtypeStruct((NC, NS, RPS, D), jnp.float32),
    mesh=_MESH,
    compiler_params=_SC_PARAMS,
    scratch_types=[
        pltpu.VMEM((GC, CH), jnp.int32),
        pltpu.VMEM((GC, CH), jnp.int32),
        pltpu.VMEM((GC, CH), jnp.float32),
        pltpu.VMEM((CH, D), jnp.float32),
        pltpu.VMEM((CH, D), jnp.float32),
        pltpu.VMEM_SHARED((N_PAD, D), jnp.float32),
        pltpu.SemaphoreType.DMA,
        pltpu.SemaphoreType.DMA,
        pltpu.SemaphoreType.DMA,
        pltpu.SemaphoreType.DMA,
    ],
)
def _msg_kernel(h_hbm, row_hbm, col_hbm, ew_hbm, out_hbm,
                row_v, col_v, ew_v, buf_a, buf_b, acc_sh,
                gsem_a, gsem_b, ssem_a, ssem_b):
    cid = lax.axis_index("c")
    sid = lax.axis_index("s")
    wid = _wid()

    zero16 = jnp.zeros((16,), jnp.float32)

    def zrow(i, carry):
        for j in range(D // 16):
            buf_a[i, pl.ds(j * 16, 16)] = zero16
        return carry

    lax.fori_loop(0, CH, zrow, 0)
    for q in range(RPS // CH):
        pltpu.sync_copy(buf_a, acc_sh.at[pl.ds(sid * RPS + q * CH, CH)])
    plsc.subcore_barrier()

    def gather_start(t, buf, sem):
        pltpu.make_async_copy(h_hbm.at[row_v.at[t]], buf, sem).start()

    def gather_wait(t, buf, sem):
        pltpu.make_async_copy(h_hbm.at[row_v.at[t]], buf, sem).wait()

    def scatter_drain(t, buf, sem):
        pltpu.make_async_copy(buf, acc_sh.at[col_v.at[t]], sem).wait()

    def scale_part(buf, t, g0, g1):
        for gg in range(g0, g1):
            nv = ew_v[t, pl.ds(gg * 16, 16)]
            for k in range(16):
                s = nv[k]
                r = gg * 16 + k
                for j in range(D // 16):
                    sl = pl.ds(j * 16, 16)
                    buf[r, sl] = buf[r, sl] * s

    def scatter_start(t, buf, sem):
        pltpu.make_async_copy(buf, acc_sh.at[col_v.at[t]], sem).start(add=True)

    def group(g, carry):
        pltpu.sync_copy(row_hbm.at[wid, g], row_v)
        pltpu.sync_copy(col_hbm.at[wid, g], col_v)
        pltpu.sync_copy(ew_hbm.at[wid, g], ew_v)

        gather_start(0, buf_a, gsem_a)

        def pair(i, c2):
            t0 = 2 * i
            t1 = t0 + 1
            gather_wait(t0, buf_a, gsem_a)

            @pl.when(i > 0)
            def _():
                scatter_drain(t0 - 1, buf_b, ssem_b)

            gather_start(t1, buf_b, gsem_b)
            scale_part(buf_a, t0, 0, NGG)
            scatter_start(t0, buf_a, ssem_a)
            gather_wait(t1, buf_b, gsem_b)
            scatter_drain(t0, buf_a, ssem_a)
            gather_start(t1 + 1, buf_a, gsem_a)
            scale_part(buf_b, t1, 0, NGG)
            scatter_start(t1, buf_b, ssem_b)
            return c2

        lax.fori_loop(0, GC // 2, pair, 0)
        tl = GC - 1
        gather_wait(tl, buf_a, gsem_a)
        scatter_drain(tl - 1, buf_b, ssem_b)
        scale_part(buf_a, tl, 0, NGG)
        scatter_start(tl, buf_a, ssem_a)
        scatter_drain(tl, buf_a, ssem_a)
        return carry

    lax.fori_loop(0, NG, group, 0)
    plsc.subcore_barrier()

    for q in range(RPS // CH):
        pltpu.sync_copy(acc_sh.at[pl.ds(sid * RPS + q * CH, CH)], buf_a)
        pltpu.sync_copy(buf_a, out_hbm.at[cid, sid, pl.ds(q * CH, CH)])


def _dis_body(degp_ref, dis_ref):
    deg = degp_ref[0] + degp_ref[1] + 1.0
    dis_ref[...] = lax.rsqrt(deg)


def _dis_call(degp):
    degp2 = degp.reshape(NC, N_PAD)
    return pl.pallas_call(
        _dis_body,
        out_shape=jax.ShapeDtypeStruct((N_PAD,), jnp.float32),
    )(degp2)


def _matmul_body(x_ref, w_ref, dis_ref, o_ref):
    o_ref[...] = jnp.dot(x_ref[...], w_ref[...],
                         preferred_element_type=jnp.float32) * dis_ref[...]


def _matmul_call(x, w, dis_col):
    bm = 1000
    grid = N_NODES // bm
    return pl.pallas_call(
        _matmul_body,
        grid=(grid,),
        in_specs=[
            pl.BlockSpec((bm, D), lambda i: (i, 0)),
            pl.BlockSpec((D, D), lambda i: (0, 0)),
            pl.BlockSpec((bm, 1), lambda i: (i, 0)),
        ],
        out_specs=pl.BlockSpec((bm, D), lambda i: (i, 0)),
        out_shape=jax.ShapeDtypeStruct((N_NODES, D), jnp.float32),
    )(x, w, dis_col)


def _combine_mm_body(m0_ref, m1_ref, h_ref, dis_ref, b_ref, a_ref, w_ref,
                     o_ref):
    dis = dis_ref[...]
    z = (m0_ref[...] + m1_ref[...] + h_ref[...]) * dis + b_ref[...]
    z = jnp.where(z > 0, z, a_ref[...] * z)
    o_ref[...] = jnp.dot(z, w_ref[...],
                         preferred_element_type=jnp.float32) * dis


def _combine_body(m0_ref, m1_ref, h_ref, dis_ref, b_ref, a_ref, o_ref):
    z = (m0_ref[...] + m1_ref[...] + h_ref[...]) * dis_ref[...] + b_ref[...]
    o_ref[...] = jnp.where(z > 0, z, a_ref[...] * z)


def _combine_call(m0, m1, h, dis_col, b, a, w=None):
    bm = 1000
    grid = N_NODES // bm
    node_spec = pl.BlockSpec((bm, D), lambda i: (i, 0))
    vec_spec = pl.BlockSpec((1, D), lambda i: (0, 0))
    in_specs = [node_spec, node_spec, node_spec,
                pl.BlockSpec((bm, 1), lambda i: (i, 0)),
                vec_spec, vec_spec]
    args = [m0, m1, h, dis_col, b.reshape(1, D), a.reshape(1, D)]
    if w is not None:
        in_specs.append(pl.BlockSpec((D, D), lambda i: (0, 0)))
        args.append(w)
        body = _combine_mm_body
    else:
        body = _combine_body
    return pl.pallas_call(
        body,
        grid=(grid,),
        in_specs=in_specs,
        out_specs=node_spec,
        out_shape=jax.ShapeDtypeStruct((N_NODES, D), jnp.float32),
    )(*args)


def kernel(x, edge_index, edge_weight, W1, b1, a1, W2, b2, a2):
    ei = edge_index.astype(jnp.int32)
    row3 = ei[0].reshape(NW, NCHUNK, CH)
    col3 = ei[1].reshape(NW, NCHUNK, CH)
    ew3 = edge_weight.reshape(NW, NCHUNK, CH)
    row4 = row3.reshape(NW, NG, GC, CH)
    col4 = col3.reshape(NW, NG, GC, CH)
    ew4 = ew3.reshape(NW, NG, GC, CH)

    degp = _deg_kernel(col3, ew3)
    dis_flat = _dis_call(degp)
    dis_col = dis_flat[:N_NODES].reshape(N_NODES, 1)

    hd1 = _matmul_call(x, W1, dis_col)
    m1 = _msg_kernel(hd1, row4, col4, ew4).reshape(NC, N_PAD, D)[:, :N_NODES]
    hd2 = _combine_call(m1[0], m1[1], hd1, dis_col, b1, a1, w=W2)

    m2 = _msg_kernel(hd2, row4, col4, ew4).reshape(NC, N_PAD, D)[:, :N_NODES]
    out = _combine_call(m2[0], m2[1], hd2, dis_col, b2, a2)
    return out
